# baseline (speedup 1.0000x reference)
import jax
import jax.numpy as jnp
from jax.experimental import pallas as pl
from jax.experimental.pallas import tpu as pltpu

_GRID_SHAPE = (6, 64)
_BATCH = 32
_DIM = 512
_L = _GRID_SHAPE[0]
_S = _GRID_SHAPE[1]
_NUM_CELLS = _L * _S
_C = 8
_NC = _S // _C
_NUM_K = (_L + _NC - 1 + 2) // 2


def _body(x_ref, w_ref, b_ref, out0_ref, out1_ref,
          buf0, buf1, sem0, sem1):
    k = pl.program_id(0)

    def wait_phase(d, slot):
        for l in range(_L):
            c = d - l
            st = 2 * l + slot

            @pl.when((c >= 2) & (c <= _NC + 1))
            def _(l=l, c=c, st=st):
                row = l * _S + _C * (c - 2)
                pltpu.make_async_copy(buf0.at[st], out0_ref.at[pl.ds(row, _C)],
                                      sem0.at[st]).wait()
                pltpu.make_async_copy(buf1.at[st], out1_ref.at[pl.ds(row, _C)],
                                      sem1.at[st]).wait()

    def cells(d, i, parity, src, dst, first):
        for l in range(_L):
            c = d - l
            std = 2 * l + dst
            sts = 2 * l + src

            @pl.when((c >= 0) & (c <= _NC - 1))
            def _(l=l, c=c, std=std, sts=sts):
                if l == 0:
                    d0 = x_ref[0, _C * parity + i]
                else:
                    d0 = buf0[2 * (l - 1) + src, i]
                if first:
                    d1 = jnp.where(c == 0, 0.0, buf1[sts, _C - 1])
                else:
                    d1 = buf1[std, i - 1]
                g0 = jnp.tanh(b_ref[l, 0] + d0 * w_ref[l, 0, 0]
                              + d1 * w_ref[l, 0, 1])
                g1 = jnp.tanh(b_ref[l, 1] + d0 * w_ref[l, 1, 0]
                              + d1 * w_ref[l, 1, 1])
                buf0[std, i] = g0
                buf1[std, i] = g1

    def diag(d, parity, src, dst):
        wait_phase(d, dst)
        cells(d, 0, parity, src, dst, True)

        def step(i, carry):
            cells(d, i, parity, src, dst, False)
            return carry

        jax.lax.fori_loop(1, _C, step, 0)

        for l in range(_L):
            c = d - l
            st = 2 * l + dst

            @pl.when((c >= 0) & (c <= _NC - 1))
            def _(l=l, c=c, st=st):
                row = l * _S + _C * c
                pltpu.make_async_copy(buf0.at[st], out0_ref.at[pl.ds(row, _C)],
                                      sem0.at[st]).start()
                pltpu.make_async_copy(buf1.at[st], out1_ref.at[pl.ds(row, _C)],
                                      sem1.at[st]).start()

    diag(2 * k, 0, 1, 0)
    diag(2 * k + 1, 1, 0, 1)

    @pl.when(k == _NUM_K - 1)
    def _drain():
        wait_phase(2 * k + 2, 0)
        wait_phase(2 * k + 3, 1)


def kernel(x, w, b):
    n_xblk = _S // (2 * _C)
    x4 = jnp.transpose(x, (1, 0, 2)).reshape(n_xblk, 2 * _C, _BATCH, _DIM)
    tile = lambda: pltpu.VMEM((2 * _L, _C, _BATCH, _DIM), x.dtype)
    out0, out1 = pl.pallas_call(
        _body,
        grid=(_NUM_K,),
        in_specs=[
            pl.BlockSpec((1, 2 * _C, _BATCH, _DIM),
                         lambda k: (jnp.minimum(k, n_xblk - 1), 0, 0, 0)),
            pl.BlockSpec(w.shape, lambda k: (0, 0, 0, 0)),
            pl.BlockSpec(b.shape, lambda k: (0, 0, 0)),
        ],
        out_specs=[
            pl.BlockSpec(memory_space=pl.ANY),
            pl.BlockSpec(memory_space=pl.ANY),
        ],
        out_shape=[
            jax.ShapeDtypeStruct((_NUM_CELLS, _BATCH, _DIM), x.dtype),
            jax.ShapeDtypeStruct((_NUM_CELLS, _BATCH, _DIM), x.dtype),
        ],
        scratch_shapes=[
            tile(), tile(),
            pltpu.SemaphoreType.DMA((2 * _L,)),
            pltpu.SemaphoreType.DMA((2 * _L,)),
        ],
        compiler_params=pltpu.CompilerParams(
            dimension_semantics=("arbitrary",),
        ),
    )(x4, w, b)
    return (out0, out1)


# speedup vs baseline: 1.0228x; 1.0228x over previous
import jax
import jax.numpy as jnp
from jax.experimental import pallas as pl
from jax.experimental.pallas import tpu as pltpu

_GRID_SHAPE = (6, 64)
_BATCH = 32
_DIM = 512
_L = _GRID_SHAPE[0]
_S = _GRID_SHAPE[1]
_NUM_CELLS = _L * _S
_C = 16
_NC = _S // _C
_NUM_K = (_L + _NC - 1 + 2) // 2


def _body(x_ref, w_ref, b_ref, out0_ref, out1_ref,
          buf0, buf1, sem0, sem1):
    k = pl.program_id(0)

    def wait_phase(d, slot):
        for l in range(_L):
            c = d - l
            st = 2 * l + slot

            @pl.when((c >= 2) & (c <= _NC + 1))
            def _(l=l, c=c, st=st):
                row = l * _S + _C * (c - 2)
                pltpu.make_async_copy(buf0.at[st], out0_ref.at[pl.ds(row, _C)],
                                      sem0.at[st]).wait()
                pltpu.make_async_copy(buf1.at[st], out1_ref.at[pl.ds(row, _C)],
                                      sem1.at[st]).wait()

    def cells(d, i, parity, src, dst, first):
        for l in range(_L):
            c = d - l
            std = 2 * l + dst
            sts = 2 * l + src

            @pl.when((c >= 0) & (c <= _NC - 1))
            def _(l=l, c=c, std=std, sts=sts):
                if l == 0:
                    d0 = x_ref[0, _C * parity + i]
                else:
                    d0 = buf0[2 * (l - 1) + src, i]
                if first:
                    d1 = jnp.where(c == 0, 0.0, buf1[sts, _C - 1])
                else:
                    d1 = buf1[std, i - 1]
                g0 = jnp.tanh(b_ref[l, 0] + d0 * w_ref[l, 0, 0]
                              + d1 * w_ref[l, 0, 1])
                g1 = jnp.tanh(b_ref[l, 1] + d0 * w_ref[l, 1, 0]
                              + d1 * w_ref[l, 1, 1])
                buf0[std, i] = g0
                buf1[std, i] = g1

    def diag(d, parity, src, dst):
        wait_phase(d, dst)
        cells(d, 0, parity, src, dst, True)
        for i in range(1, _C):
            cells(d, i, parity, src, dst, False)

        for l in range(_L):
            c = d - l
            st = 2 * l + dst

            @pl.when((c >= 0) & (c <= _NC - 1))
            def _(l=l, c=c, st=st):
                row = l * _S + _C * c
                pltpu.make_async_copy(buf0.at[st], out0_ref.at[pl.ds(row, _C)],
                                      sem0.at[st]).start()
                pltpu.make_async_copy(buf1.at[st], out1_ref.at[pl.ds(row, _C)],
                                      sem1.at[st]).start()

    diag(2 * k, 0, 1, 0)
    diag(2 * k + 1, 1, 0, 1)

    @pl.when(k == _NUM_K - 1)
    def _drain():
        wait_phase(2 * k + 2, 0)
        wait_phase(2 * k + 3, 1)


def kernel(x, w, b):
    x4 = jnp.transpose(x, (1, 0, 2)).reshape(2, _S // 2, _BATCH, _DIM)
    tile = lambda: pltpu.VMEM((2 * _L, _C, _BATCH, _DIM), x.dtype)
    out0, out1 = pl.pallas_call(
        _body,
        grid=(_NUM_K,),
        in_specs=[
            pl.BlockSpec((1, _S // 2, _BATCH, _DIM),
                         lambda k: (jnp.minimum(k, 1), 0, 0, 0)),
            pl.BlockSpec(w.shape, lambda k: (0, 0, 0, 0)),
            pl.BlockSpec(b.shape, lambda k: (0, 0, 0)),
        ],
        out_specs=[
            pl.BlockSpec(memory_space=pl.ANY),
            pl.BlockSpec(memory_space=pl.ANY),
        ],
        out_shape=[
            jax.ShapeDtypeStruct((_NUM_CELLS, _BATCH, _DIM), x.dtype),
            jax.ShapeDtypeStruct((_NUM_CELLS, _BATCH, _DIM), x.dtype),
        ],
        scratch_shapes=[
            tile(), tile(),
            pltpu.SemaphoreType.DMA((2 * _L,)),
            pltpu.SemaphoreType.DMA((2 * _L,)),
        ],
        compiler_params=pltpu.CompilerParams(
            dimension_semantics=("arbitrary",),
        ),
    )(x4, w, b)
    return (out0, out1)


# speedup vs baseline: 1.0948x; 1.0704x over previous
import jax
import jax.numpy as jnp
from jax.experimental import pallas as pl
from jax.experimental.pallas import tpu as pltpu

_GRID_SHAPE = (6, 64)
_BATCH = 32
_DIM = 512
_L = _GRID_SHAPE[0]
_S = _GRID_SHAPE[1]
_NUM_CELLS = _L * _S
_C = 16
_NC = _S // _C
_ND = _L + _NC - 1
_NUM_K = (_ND + 1) // 2


def _body(x_ref, w_ref, b_ref, out0_ref, out1_ref,
          buf0, buf1, sem0, sem1):
    k = pl.program_id(0)

    def copies(l, c, slot):
        row = l * _S + _C * c
        st = 2 * l + slot
        return (
            pltpu.make_async_copy(buf0.at[st], out0_ref.at[pl.ds(row, _C)],
                                  sem0.at[st]),
            pltpu.make_async_copy(buf1.at[st], out1_ref.at[pl.ds(row, _C)],
                                  sem1.at[st]),
        )

    def active(d):
        return [(l, d - l) for l in range(_L) if 0 <= d - l <= _NC - 1]

    def diag(d, src, dst):
        parity = d & 1
        for l, c in active(d - 2):
            c0, c1 = copies(l, c, dst)
            c0.wait()
            c1.wait()

        for i in range(_C):
            for l, c in active(d):
                if l == 0:
                    d0 = x_ref[0, _C * parity + i]
                else:
                    d0 = buf0[2 * (l - 1) + src, i]
                if i > 0:
                    d1 = buf1[2 * l + dst, i - 1]
                elif c == 0:
                    d1 = jnp.zeros((_BATCH, _DIM), dtype=x_ref.dtype)
                else:
                    d1 = buf1[2 * l + src, _C - 1]
                g0 = jnp.tanh(b_ref[l, 0] + d0 * w_ref[l, 0, 0]
                              + d1 * w_ref[l, 0, 1])
                g1 = jnp.tanh(b_ref[l, 1] + d0 * w_ref[l, 1, 0]
                              + d1 * w_ref[l, 1, 1])
                buf0[2 * l + dst, i] = g0
                buf1[2 * l + dst, i] = g1

        for l, c in active(d):
            c0, c1 = copies(l, c, dst)
            c0.start()
            c1.start()

    for kk in range(_NUM_K):
        @pl.when(k == kk)
        def _(kk=kk):
            diag(2 * kk, 1, 0)
            diag(2 * kk + 1, 0, 1)
            if kk == _NUM_K - 1:
                for dd, slot in ((2 * kk, 0), (2 * kk + 1, 1)):
                    for l, c in active(dd):
                        c0, c1 = copies(l, c, slot)
                        c0.wait()
                        c1.wait()


def kernel(x, w, b):
    x4 = jnp.transpose(x, (1, 0, 2)).reshape(_S // (2 * _C), 2 * _C,
                                             _BATCH, _DIM)
    n_xblk = _S // (2 * _C)
    tile = lambda: pltpu.VMEM((2 * _L, _C, _BATCH, _DIM), x.dtype)
    out0, out1 = pl.pallas_call(
        _body,
        grid=(_NUM_K,),
        in_specs=[
            pl.BlockSpec((1, 2 * _C, _BATCH, _DIM),
                         lambda k: (jnp.minimum(k, n_xblk - 1), 0, 0, 0)),
            pl.BlockSpec(w.shape, lambda k: (0, 0, 0, 0)),
            pl.BlockSpec(b.shape, lambda k: (0, 0, 0)),
        ],
        out_specs=[
            pl.BlockSpec(memory_space=pl.ANY),
            pl.BlockSpec(memory_space=pl.ANY),
        ],
        out_shape=[
            jax.ShapeDtypeStruct((_NUM_CELLS, _BATCH, _DIM), x.dtype),
            jax.ShapeDtypeStruct((_NUM_CELLS, _BATCH, _DIM), x.dtype),
        ],
        scratch_shapes=[
            tile(), tile(),
            pltpu.SemaphoreType.DMA((2 * _L,)),
            pltpu.SemaphoreType.DMA((2 * _L,)),
        ],
        compiler_params=pltpu.CompilerParams(
            dimension_semantics=("arbitrary",),
        ),
    )(x4, w, b)
    return (out0, out1)


# speedup vs baseline: 1.0965x; 1.0016x over previous
import jax
import jax.numpy as jnp
from jax.experimental import pallas as pl
from jax.experimental.pallas import tpu as pltpu

_GRID_SHAPE = (6, 64)
_BATCH = 32
_DIM = 512
_L = _GRID_SHAPE[0]
_S = _GRID_SHAPE[1]
_NUM_CELLS = _L * _S
_C = 8
_NC = _S // _C
_ND = _L + _NC - 1
_NUM_K = (_ND + 1) // 2


def _body(x_ref, w_ref, b_ref, out0_ref, out1_ref,
          buf0, buf1, sem0, sem1):
    k = pl.program_id(0)

    def copies(l, c, slot):
        row = l * _S + _C * c
        st = 2 * l + slot
        return (
            pltpu.make_async_copy(buf0.at[st], out0_ref.at[pl.ds(row, _C)],
                                  sem0.at[st]),
            pltpu.make_async_copy(buf1.at[st], out1_ref.at[pl.ds(row, _C)],
                                  sem1.at[st]),
        )

    def active(d):
        return [(l, d - l) for l in range(_L) if 0 <= d - l <= _NC - 1]

    def diag(d, src, dst):
        parity = d & 1
        for l, c in active(d - 2):
            c0, c1 = copies(l, c, dst)
            c0.wait()
            c1.wait()

        for i in range(_C):
            for l, c in active(d):
                if l == 0:
                    d0 = x_ref[0, _C * parity + i]
                else:
                    d0 = buf0[2 * (l - 1) + src, i]
                if i > 0:
                    d1 = buf1[2 * l + dst, i - 1]
                elif c == 0:
                    d1 = jnp.zeros((_BATCH, _DIM), dtype=x_ref.dtype)
                else:
                    d1 = buf1[2 * l + src, _C - 1]
                g0 = jnp.tanh(b_ref[l, 0] + d0 * w_ref[l, 0, 0]
                              + d1 * w_ref[l, 0, 1])
                g1 = jnp.tanh(b_ref[l, 1] + d0 * w_ref[l, 1, 0]
                              + d1 * w_ref[l, 1, 1])
                buf0[2 * l + dst, i] = g0
                buf1[2 * l + dst, i] = g1

        for l, c in active(d):
            c0, c1 = copies(l, c, dst)
            c0.start()
            c1.start()

    for kk in range(_NUM_K):
        @pl.when(k == kk)
        def _(kk=kk):
            diag(2 * kk, 1, 0)
            diag(2 * kk + 1, 0, 1)
            if kk == _NUM_K - 1:
                for dd, slot in ((2 * kk, 0), (2 * kk + 1, 1)):
                    for l, c in active(dd):
                        c0, c1 = copies(l, c, slot)
                        c0.wait()
                        c1.wait()


def kernel(x, w, b):
    x4 = jnp.transpose(x, (1, 0, 2)).reshape(_S // (2 * _C), 2 * _C,
                                             _BATCH, _DIM)
    n_xblk = _S // (2 * _C)
    tile = lambda: pltpu.VMEM((2 * _L, _C, _BATCH, _DIM), x.dtype)
    out0, out1 = pl.pallas_call(
        _body,
        grid=(_NUM_K,),
        in_specs=[
            pl.BlockSpec((1, 2 * _C, _BATCH, _DIM),
                         lambda k: (jnp.minimum(k, n_xblk - 1), 0, 0, 0)),
            pl.BlockSpec(w.shape, lambda k: (0, 0, 0, 0)),
            pl.BlockSpec(b.shape, lambda k: (0, 0, 0)),
        ],
        out_specs=[
            pl.BlockSpec(memory_space=pl.ANY),
            pl.BlockSpec(memory_space=pl.ANY),
        ],
        out_shape=[
            jax.ShapeDtypeStruct((_NUM_CELLS, _BATCH, _DIM), x.dtype),
            jax.ShapeDtypeStruct((_NUM_CELLS, _BATCH, _DIM), x.dtype),
        ],
        scratch_shapes=[
            tile(), tile(),
            pltpu.SemaphoreType.DMA((2 * _L,)),
            pltpu.SemaphoreType.DMA((2 * _L,)),
        ],
        compiler_params=pltpu.CompilerParams(
            dimension_semantics=("arbitrary",),
        ),
    )(x4, w, b)
    return (out0, out1)
